# Initial kernel scaffold; baseline (speedup 1.0000x reference)
import jax
import jax.numpy as jnp
from jax.experimental import pallas as pl


def kernel(x, weight):
    raise NotImplementedError("write your pallas kernel here")


# speedup vs baseline: 1.6454x; 1.6454x over previous
import functools

import jax
import jax.numpy as jnp
from jax.experimental import pallas as pl

N_TOKENS = 32768
DIM = 768
N_EXPERTS = 64
TOPK = 8
N_GROUPS = 8
GROUP_SIZE = N_EXPERTS // N_GROUPS
TOPK_GROUPS = 4

TILE = 256

NEG_INF = float("-inf")


def _gate_kernel(x_ref, wt_ref, w_out_ref, i_out_ref):
    t = x_ref.shape[0]
    scores = jnp.dot(x_ref[...], wt_ref[...], preferred_element_type=jnp.float32)
    smax = jnp.max(scores, axis=1, keepdims=True)
    e = jnp.exp(scores - smax)
    p = e / jnp.sum(e, axis=1, keepdims=True)

    lane = jax.lax.broadcasted_iota(jnp.int32, (t, N_EXPERTS), 1)
    glane = jax.lax.broadcasted_iota(jnp.int32, (t, N_GROUPS), 1)
    group_of = lane // GROUP_SIZE

    gs = jnp.max(p.reshape(t, N_GROUPS, GROUP_SIZE), axis=-1)

    keep = jnp.zeros((t, N_EXPERTS), dtype=jnp.bool_)
    for _ in range(TOPK_GROUPS):
        gm = jnp.max(gs, axis=1, keepdims=True)
        gidx = jnp.min(jnp.where(gs == gm, glane, N_GROUPS), axis=1, keepdims=True)
        keep = keep | (group_of == gidx)
        gs = jnp.where(glane == gidx, NEG_INF, gs)

    sm = jnp.where(keep, p, NEG_INF)

    vals = []
    idxs = []
    for _ in range(TOPK):
        m = jnp.max(sm, axis=1, keepdims=True)
        idx = jnp.min(jnp.where(sm == m, lane, N_EXPERTS), axis=1, keepdims=True)
        vals.append(m)
        idxs.append(idx)
        sm = jnp.where(lane == idx, NEG_INF, sm)

    w_out_ref[...] = jnp.concatenate(vals, axis=1)
    i_out_ref[...] = jnp.concatenate(idxs, axis=1)


@jax.jit
def kernel(x, weight):
    n = x.shape[0]
    wt = weight.T
    grid = (n // TILE,)
    w_out, i_out = pl.pallas_call(
        _gate_kernel,
        grid=grid,
        in_specs=[
            pl.BlockSpec((TILE, DIM), lambda i: (i, 0)),
            pl.BlockSpec((DIM, N_EXPERTS), lambda i: (0, 0)),
        ],
        out_specs=[
            pl.BlockSpec((TILE, TOPK), lambda i: (i, 0)),
            pl.BlockSpec((TILE, TOPK), lambda i: (i, 0)),
        ],
        out_shape=[
            jax.ShapeDtypeStruct((n, TOPK), jnp.float32),
            jax.ShapeDtypeStruct((n, TOPK), jnp.int32),
        ],
    )(x, wt)
    return w_out, i_out


# speedup vs baseline: 5.7855x; 3.5162x over previous
import jax
import jax.numpy as jnp
from jax.experimental import pallas as pl

N_TOKENS = 32768
DIM = 768
N_EXPERTS = 64
TOPK = 8
N_GROUPS = 8
GROUP_SIZE = N_EXPERTS // N_GROUPS
TOPK_GROUPS = 4

TILE = 256

NEG_INF = float("-inf")


def _gate_kernel(x_ref, wt_ref, w_out_ref, i_out_ref):
    t = x_ref.shape[0]
    scores = jnp.dot(x_ref[...], wt_ref[...], preferred_element_type=jnp.float32)
    s = scores.T

    smax = jnp.max(s, axis=0, keepdims=True)
    denom = jnp.sum(jnp.exp(s - smax), axis=0, keepdims=True)

    eidx = jax.lax.broadcasted_iota(jnp.int32, (N_EXPERTS, t), 0)
    gidx_iota = jax.lax.broadcasted_iota(jnp.int32, (N_GROUPS, t), 0)
    group_of = eidx // GROUP_SIZE

    gs = jnp.max(s.reshape(N_GROUPS, GROUP_SIZE, t), axis=1)

    keep = jnp.zeros((N_EXPERTS, t), dtype=jnp.bool_)
    for k in range(TOPK_GROUPS):
        gm = jnp.max(gs, axis=0, keepdims=True)
        gsel = jnp.min(jnp.where(gs == gm, gidx_iota, N_GROUPS), axis=0,
                       keepdims=True)
        keep = keep | (group_of == gsel)
        if k != TOPK_GROUPS - 1:
            gs = jnp.where(gidx_iota == gsel, NEG_INF, gs)

    sm = jnp.where(keep, s, NEG_INF)

    vals = []
    idxs = []
    for k in range(TOPK):
        m = jnp.max(sm, axis=0, keepdims=True)
        idx = jnp.min(jnp.where(sm == m, eidx, N_EXPERTS), axis=0,
                      keepdims=True)
        vals.append(m)
        idxs.append(idx)
        if k != TOPK - 1:
            sm = jnp.where(eidx == idx, NEG_INF, sm)

    w_t = jnp.exp(jnp.concatenate(vals, axis=0) - smax) / denom
    i_t = jnp.concatenate(idxs, axis=0)

    w_out_ref[...] = w_t.T
    i_out_ref[...] = i_t.T


@jax.jit
def kernel(x, weight):
    n = x.shape[0]
    wt = weight.T
    grid = (n // TILE,)
    w_out, i_out = pl.pallas_call(
        _gate_kernel,
        grid=grid,
        in_specs=[
            pl.BlockSpec((TILE, DIM), lambda i: (i, 0)),
            pl.BlockSpec((DIM, N_EXPERTS), lambda i: (0, 0)),
        ],
        out_specs=[
            pl.BlockSpec((TILE, TOPK), lambda i: (i, 0)),
            pl.BlockSpec((TILE, TOPK), lambda i: (i, 0)),
        ],
        out_shape=[
            jax.ShapeDtypeStruct((n, TOPK), jnp.float32),
            jax.ShapeDtypeStruct((n, TOPK), jnp.int32),
        ],
    )(x, wt)
    return w_out, i_out


# speedup vs baseline: 8.2117x; 1.4194x over previous
import jax
import jax.numpy as jnp
from jax.experimental import pallas as pl

N_TOKENS = 32768
DIM = 768
N_EXPERTS = 64
TOPK = 8
N_GROUPS = 8
GROUP_SIZE = N_EXPERTS // N_GROUPS
TOPK_GROUPS = 4

TILE = 512

NEG_INF = float("-inf")


def _gate_kernel(x_ref, wt_ref, w_out_ref, i_out_ref):
    t = x_ref.shape[0]
    scores = jnp.dot(x_ref[...], wt_ref[...], preferred_element_type=jnp.float32)
    s = scores.T

    smax = jnp.max(s, axis=0, keepdims=True)
    denom = jnp.sum(jnp.exp(s - smax), axis=0, keepdims=True)

    eidx = jax.lax.broadcasted_iota(jnp.int32, (N_EXPERTS, t), 0)
    gidx_iota = jax.lax.broadcasted_iota(jnp.int32, (N_GROUPS, t), 0)
    group_of = eidx // GROUP_SIZE

    gs = jnp.max(s.reshape(N_GROUPS, GROUP_SIZE, t), axis=1)

    keep = jnp.zeros((N_EXPERTS, t), dtype=jnp.bool_)
    for k in range(TOPK_GROUPS):
        gm = jnp.max(gs, axis=0, keepdims=True)
        gsel = jnp.min(jnp.where(gs == gm, gidx_iota, N_GROUPS), axis=0,
                       keepdims=True)
        keep = keep | (group_of == gsel)
        if k != TOPK_GROUPS - 1:
            gs = jnp.where(gidx_iota == gsel, NEG_INF, gs)

    sm = jnp.where(keep, s, NEG_INF)

    vals = []
    idxs = []
    for k in range(TOPK):
        m = jnp.max(sm, axis=0, keepdims=True)
        idx = jnp.min(jnp.where(sm == m, eidx, N_EXPERTS), axis=0,
                      keepdims=True)
        vals.append(m)
        idxs.append(idx)
        if k != TOPK - 1:
            sm = jnp.where(eidx == idx, NEG_INF, sm)

    w_t = jnp.exp(jnp.concatenate(vals, axis=0) - smax) / denom
    i_t = jnp.concatenate(idxs, axis=0)

    w_out_ref[...] = w_t.T
    i_out_ref[...] = i_t.T


@jax.jit
def kernel(x, weight):
    n = x.shape[0]
    wt = weight.T
    grid = (n // TILE,)
    w_out, i_out = pl.pallas_call(
        _gate_kernel,
        grid=grid,
        in_specs=[
            pl.BlockSpec((TILE, DIM), lambda i: (i, 0)),
            pl.BlockSpec((DIM, N_EXPERTS), lambda i: (0, 0)),
        ],
        out_specs=[
            pl.BlockSpec((TILE, TOPK), lambda i: (i, 0)),
            pl.BlockSpec((TILE, TOPK), lambda i: (i, 0)),
        ],
        out_shape=[
            jax.ShapeDtypeStruct((n, TOPK), jnp.float32),
            jax.ShapeDtypeStruct((n, TOPK), jnp.int32),
        ],
    )(x, wt)
    return w_out, i_out


# speedup vs baseline: 10.3992x; 1.2664x over previous
import jax
import jax.numpy as jnp
from jax.experimental import pallas as pl

N_TOKENS = 32768
DIM = 768
N_EXPERTS = 64
TOPK = 8
N_GROUPS = 8
GROUP_SIZE = N_EXPERTS // N_GROUPS
TOPK_GROUPS = 4

TILE = 1024

NEG_INF = float("-inf")


def _gate_kernel(x_ref, wt_ref, w_out_ref, i_out_ref):
    t = x_ref.shape[0]
    scores = jnp.dot(x_ref[...], wt_ref[...], preferred_element_type=jnp.float32)
    s = scores.T

    smax = jnp.max(s, axis=0, keepdims=True)
    denom = jnp.sum(jnp.exp(s - smax), axis=0, keepdims=True)

    eidx = jax.lax.broadcasted_iota(jnp.int32, (N_EXPERTS, t), 0)
    gidx_iota = jax.lax.broadcasted_iota(jnp.int32, (N_GROUPS, t), 0)
    group_of = eidx // GROUP_SIZE

    gs = jnp.max(s.reshape(N_GROUPS, GROUP_SIZE, t), axis=1)

    keep = jnp.zeros((N_EXPERTS, t), dtype=jnp.bool_)
    for k in range(TOPK_GROUPS):
        gm = jnp.max(gs, axis=0, keepdims=True)
        gsel = jnp.min(jnp.where(gs == gm, gidx_iota, N_GROUPS), axis=0,
                       keepdims=True)
        keep = keep | (group_of == gsel)
        if k != TOPK_GROUPS - 1:
            gs = jnp.where(gidx_iota == gsel, NEG_INF, gs)

    sm = jnp.where(keep, s, NEG_INF)

    vals = []
    idxs = []
    for k in range(TOPK):
        m = jnp.max(sm, axis=0, keepdims=True)
        idx = jnp.min(jnp.where(sm == m, eidx, N_EXPERTS), axis=0,
                      keepdims=True)
        vals.append(m)
        idxs.append(idx)
        if k != TOPK - 1:
            sm = jnp.where(eidx == idx, NEG_INF, sm)

    w_t = jnp.exp(jnp.concatenate(vals, axis=0) - smax) / denom
    i_t = jnp.concatenate(idxs, axis=0)

    w_out_ref[...] = w_t.T
    i_out_ref[...] = i_t.T


@jax.jit
def kernel(x, weight):
    n = x.shape[0]
    wt = weight.T
    grid = (n // TILE,)
    w_out, i_out = pl.pallas_call(
        _gate_kernel,
        grid=grid,
        in_specs=[
            pl.BlockSpec((TILE, DIM), lambda i: (i, 0)),
            pl.BlockSpec((DIM, N_EXPERTS), lambda i: (0, 0)),
        ],
        out_specs=[
            pl.BlockSpec((TILE, TOPK), lambda i: (i, 0)),
            pl.BlockSpec((TILE, TOPK), lambda i: (i, 0)),
        ],
        out_shape=[
            jax.ShapeDtypeStruct((n, TOPK), jnp.float32),
            jax.ShapeDtypeStruct((n, TOPK), jnp.int32),
        ],
    )(x, wt)
    return w_out, i_out


# speedup vs baseline: 11.5951x; 1.1150x over previous
import jax
import jax.numpy as jnp
from jax.experimental import pallas as pl

N_TOKENS = 32768
DIM = 768
N_EXPERTS = 64
TOPK = 8
N_GROUPS = 8
GROUP_SIZE = N_EXPERTS // N_GROUPS
TOPK_GROUPS = 4

TILE = 2048

NEG_INF = float("-inf")


def _gate_kernel(x_ref, wt_ref, w_out_ref, i_out_ref):
    t = x_ref.shape[0]
    scores = jnp.dot(x_ref[...], wt_ref[...], preferred_element_type=jnp.float32)
    s = scores.T

    smax = jnp.max(s, axis=0, keepdims=True)
    denom = jnp.sum(jnp.exp(s - smax), axis=0, keepdims=True)

    eidx = jax.lax.broadcasted_iota(jnp.int32, (N_EXPERTS, t), 0)
    gidx_iota = jax.lax.broadcasted_iota(jnp.int32, (N_GROUPS, t), 0)
    group_of = eidx // GROUP_SIZE

    gs = jnp.max(s.reshape(N_GROUPS, GROUP_SIZE, t), axis=1)

    keep = jnp.zeros((N_EXPERTS, t), dtype=jnp.bool_)
    for k in range(TOPK_GROUPS):
        gm = jnp.max(gs, axis=0, keepdims=True)
        gsel = jnp.min(jnp.where(gs == gm, gidx_iota, N_GROUPS), axis=0,
                       keepdims=True)
        keep = keep | (group_of == gsel)
        if k != TOPK_GROUPS - 1:
            gs = jnp.where(gidx_iota == gsel, NEG_INF, gs)

    sm = jnp.where(keep, s, NEG_INF)

    vals = []
    idxs = []
    for k in range(TOPK):
        m = jnp.max(sm, axis=0, keepdims=True)
        idx = jnp.min(jnp.where(sm == m, eidx, N_EXPERTS), axis=0,
                      keepdims=True)
        vals.append(m)
        idxs.append(idx)
        if k != TOPK - 1:
            sm = jnp.where(eidx == idx, NEG_INF, sm)

    w_t = jnp.exp(jnp.concatenate(vals, axis=0) - smax) / denom
    i_t = jnp.concatenate(idxs, axis=0)

    w_out_ref[...] = w_t.T
    i_out_ref[...] = i_t.T


@jax.jit
def kernel(x, weight):
    n = x.shape[0]
    wt = weight.T
    grid = (n // TILE,)
    w_out, i_out = pl.pallas_call(
        _gate_kernel,
        grid=grid,
        in_specs=[
            pl.BlockSpec((TILE, DIM), lambda i: (i, 0)),
            pl.BlockSpec((DIM, N_EXPERTS), lambda i: (0, 0)),
        ],
        out_specs=[
            pl.BlockSpec((TILE, TOPK), lambda i: (i, 0)),
            pl.BlockSpec((TILE, TOPK), lambda i: (i, 0)),
        ],
        out_shape=[
            jax.ShapeDtypeStruct((n, TOPK), jnp.float32),
            jax.ShapeDtypeStruct((n, TOPK), jnp.int32),
        ],
    )(x, wt)
    return w_out, i_out


# speedup vs baseline: 12.2393x; 1.0556x over previous
import jax
import jax.numpy as jnp
from jax.experimental import pallas as pl

N_TOKENS = 32768
DIM = 768
N_EXPERTS = 64
TOPK = 8
N_GROUPS = 8
GROUP_SIZE = N_EXPERTS // N_GROUPS
TOPK_GROUPS = 4

TILE = 4096

NEG_INF = float("-inf")


def _gate_kernel(x_ref, wt_ref, w_out_ref, i_out_ref):
    t = x_ref.shape[0]
    scores = jnp.dot(x_ref[...], wt_ref[...], preferred_element_type=jnp.float32)
    s = scores.T

    smax = jnp.max(s, axis=0, keepdims=True)
    denom = jnp.sum(jnp.exp(s - smax), axis=0, keepdims=True)

    eidx = jax.lax.broadcasted_iota(jnp.int32, (N_EXPERTS, t), 0)
    gidx_iota = jax.lax.broadcasted_iota(jnp.int32, (N_GROUPS, t), 0)
    group_of = eidx // GROUP_SIZE

    gs = jnp.max(s.reshape(N_GROUPS, GROUP_SIZE, t), axis=1)

    keep = jnp.zeros((N_EXPERTS, t), dtype=jnp.bool_)
    for k in range(TOPK_GROUPS):
        gm = jnp.max(gs, axis=0, keepdims=True)
        gsel = jnp.min(jnp.where(gs == gm, gidx_iota, N_GROUPS), axis=0,
                       keepdims=True)
        keep = keep | (group_of == gsel)
        if k != TOPK_GROUPS - 1:
            gs = jnp.where(gidx_iota == gsel, NEG_INF, gs)

    sm = jnp.where(keep, s, NEG_INF)

    vals = []
    idxs = []
    for k in range(TOPK):
        m = jnp.max(sm, axis=0, keepdims=True)
        idx = jnp.min(jnp.where(sm == m, eidx, N_EXPERTS), axis=0,
                      keepdims=True)
        vals.append(m)
        idxs.append(idx)
        if k != TOPK - 1:
            sm = jnp.where(eidx == idx, NEG_INF, sm)

    w_t = jnp.exp(jnp.concatenate(vals, axis=0) - smax) / denom
    i_t = jnp.concatenate(idxs, axis=0)

    w_out_ref[...] = w_t.T
    i_out_ref[...] = i_t.T


@jax.jit
def kernel(x, weight):
    n = x.shape[0]
    wt = weight.T
    grid = (n // TILE,)
    w_out, i_out = pl.pallas_call(
        _gate_kernel,
        grid=grid,
        in_specs=[
            pl.BlockSpec((TILE, DIM), lambda i: (i, 0)),
            pl.BlockSpec((DIM, N_EXPERTS), lambda i: (0, 0)),
        ],
        out_specs=[
            pl.BlockSpec((TILE, TOPK), lambda i: (i, 0)),
            pl.BlockSpec((TILE, TOPK), lambda i: (i, 0)),
        ],
        out_shape=[
            jax.ShapeDtypeStruct((n, TOPK), jnp.float32),
            jax.ShapeDtypeStruct((n, TOPK), jnp.int32),
        ],
    )(x, wt)
    return w_out, i_out
